# baseline (speedup 1.0000x reference)
import functools

import numpy as np
import jax
import jax.numpy as jnp
from jax import lax
from jax.experimental import pallas as pl
from jax.experimental.pallas import tpu as pltpu
from jax.experimental.pallas import tpu_sc as plsc

N = 10000
D = 128
E = 320000
ALPHA = 0.1
BETA = float(np.log(1.5))

NC, NS = 2, 16
NW = NC * NS
CHUNK = 100
NCHUNK = 100
NPH = 4
CPP = NCHUNK // NPH
NPAD = 10112
RPS = NPAD // NS
NPC = 10240
RPSC = NPC // NS

_mesh = plsc.VectorSubcoreMesh(core_axis_name="c", subcore_axis_name="s")


@functools.partial(
    pl.kernel,
    out_type=jax.ShapeDtypeStruct((NC, NPC), jnp.float32),
    mesh=_mesh,
    scratch_types=[
        pltpu.VMEM((CPP, CHUNK), jnp.int32),
        pltpu.VMEM((128,), jnp.float32),
        pltpu.VMEM_SHARED((NPC,), jnp.float32),
    ],
)
def _sc_count(col_hbm, cnt_hbm, idx_v, ones_v, cnt_sh):
    c = lax.axis_index("c")
    s = lax.axis_index("s")
    w = c * NS + s
    for k in range(8):
        ones_v[pl.ds(k * 16, 16)] = jnp.zeros((16,), jnp.float32)
    for t in range(RPSC // 128):
        pltpu.sync_copy(ones_v, cnt_sh.at[pl.ds(s * RPSC + t * 128, 128)])
    for k in range(8):
        ones_v[pl.ds(k * 16, 16)] = jnp.ones((16,), jnp.float32)
    plsc.subcore_barrier()

    def phase(h, carry):
        pltpu.sync_copy(col_hbm.at[w, h], idx_v)

        def body(j, carry2):
            pltpu.sync_copy(ones_v.at[pl.ds(0, CHUNK)],
                            cnt_sh.at[idx_v.at[j]], add=True)
            return carry2

        lax.fori_loop(0, CPP, body, 0)
        return carry

    lax.fori_loop(0, NPH, phase, 0)
    plsc.subcore_barrier()
    pltpu.sync_copy(cnt_sh.at[pl.ds(s * RPSC, RPSC)],
                    cnt_hbm.at[c, pl.ds(s * RPSC, RPSC)])


@functools.partial(
    pl.kernel,
    out_type=jax.ShapeDtypeStruct((NC, NPAD, D), jnp.float32),
    mesh=_mesh,
    scratch_types=[
        pltpu.VMEM((CPP, CHUNK), jnp.int32),
        pltpu.VMEM((CPP, CHUNK), jnp.int32),
        pltpu.VMEM((3, CHUNK, D), jnp.float32),
        pltpu.VMEM_SHARED((NPAD, D), jnp.float32),
        pltpu.SemaphoreType.DMA((3,)),
    ],
)
def _sc_aggregate(xs_hbm, row_hbm, col_hbm, agg_hbm,
                  row_v, col_v, buf2, agg_sh, sem2):
    c = lax.axis_index("c")
    s = lax.axis_index("s")
    w = c * NS + s

    def zbody(i, carry):
        for k in range(D // 16):
            buf2[0, i, pl.ds(k * 16, 16)] = jnp.zeros((16,), jnp.float32)
        return carry

    lax.fori_loop(0, CHUNK, zbody, 0)
    for t in range(6):
        pltpu.sync_copy(buf2.at[0, pl.ds(0, 96)],
                        agg_sh.at[pl.ds(s * RPS + t * 96, 96)])
    pltpu.sync_copy(buf2.at[0, pl.ds(0, 56)],
                    agg_sh.at[pl.ds(s * RPS + 576, 56)])
    plsc.subcore_barrier()

    def phase(h, carry):
        pltpu.sync_copy(row_hbm.at[w, h], row_v)
        pltpu.sync_copy(col_hbm.at[w, h], col_v)

        def body(j, carry2):
            @pl.when(j < CPP)
            def _():
                b = j % 3
                pltpu.async_copy(xs_hbm.at[row_v.at[j]], buf2.at[b],
                                 sem2.at[b])

            @pl.when(j > 1)
            def _():
                p = (j - 2) % 3
                pltpu.make_async_copy(xs_hbm.at[row_v.at[0]],
                                      buf2.at[p], sem2.at[p]).wait()
                pltpu.sync_copy(buf2.at[p], agg_sh.at[col_v.at[j - 2]],
                                add=True)

            return carry2

        lax.fori_loop(0, CPP + 2, body, 0)
        return carry

    lax.fori_loop(0, NPH, phase, 0)
    plsc.subcore_barrier()
    pltpu.sync_copy(agg_sh.at[pl.ds(s * RPS, RPS)],
                    agg_hbm.at[c, pl.ds(s * RPS, RPS)])


def _prep_body(cnt_ref, x_ref, xs_ref, d_ref):
    dblk = lax.rsqrt(cnt_ref[0] + cnt_ref[1] + 1.0)
    d_ref[...] = dblk
    xs_ref[...] = x_ref[...] * dblk


_prep = pl.pallas_call(
    _prep_body,
    grid=(10,),
    in_specs=[pl.BlockSpec((NC, N // 10, 1), lambda i: (0, i, 0)),
              pl.BlockSpec((N // 10, D), lambda i: (i, 0))],
    out_specs=[pl.BlockSpec((N // 10, D), lambda i: (i, 0)),
               pl.BlockSpec((N // 10, 1), lambda i: (i, 0))],
    out_shape=[jax.ShapeDtypeStruct((N, D), jnp.float32),
               jax.ShapeDtypeStruct((N, 1), jnp.float32)],
)


def _combine_body(agg_ref, d_ref, x_ref, x0_ref, wp_ref, out_ref):
    dblk = d_ref[...]
    d1 = (1.0 - ALPHA) * dblk
    d2 = d1 * dblk
    b = agg_ref[0] + agg_ref[1]
    support = d1 * b + d2 * x_ref[...] + ALPHA * x0_ref[...]
    out_ref[...] = jnp.dot(support, wp_ref[...],
                           preferred_element_type=jnp.float32)


_combine = pl.pallas_call(
    _combine_body,
    grid=(10,),
    in_specs=[pl.BlockSpec((NC, N // 10, D), lambda i: (0, i, 0)),
              pl.BlockSpec((N // 10, 1), lambda i: (i, 0)),
              pl.BlockSpec((N // 10, D), lambda i: (i, 0)),
              pl.BlockSpec((N // 10, D), lambda i: (i, 0)),
              pl.BlockSpec((D, D), lambda i: (0, 0))],
    out_specs=pl.BlockSpec((N // 10, D), lambda i: (i, 0)),
    out_shape=jax.ShapeDtypeStruct((N, D), jnp.float32),
)


def kernel(x, edge_index, x0, W):
    row_p = edge_index[0].reshape(NW, NPH, CPP, CHUNK)
    col_p = edge_index[1].reshape(NW, NPH, CPP, CHUNK)

    cnt3 = _sc_count(col_p).reshape(NC, NPC, 1)
    xs, d = _prep(cnt3, x)
    agg2 = _sc_aggregate(xs, row_p, col_p)

    Wp = (1.0 - BETA) * jnp.eye(D, dtype=jnp.float32) + BETA * W
    return _combine(agg2, d, x, x0, Wp)


# speedup vs baseline: 1.0157x; 1.0157x over previous
import functools

import numpy as np
import jax
import jax.numpy as jnp
from jax import lax
from jax.experimental import pallas as pl
from jax.experimental.pallas import tpu as pltpu
from jax.experimental.pallas import tpu_sc as plsc

N = 10000
D = 128
E = 320000
ALPHA = 0.1
BETA = float(np.log(1.5))

NC, NS = 2, 16
NW = NC * NS
CHUNK = 100
NCHUNK = 100
NPH = 4
CPP = NCHUNK // NPH
NPAD = 10112
RPS = NPAD // NS
NPC = 10240
RPSC = NPC // NS

_mesh = plsc.VectorSubcoreMesh(core_axis_name="c", subcore_axis_name="s")


@functools.partial(
    pl.kernel,
    out_type=jax.ShapeDtypeStruct((NC, NPC), jnp.float32),
    mesh=_mesh,
    scratch_types=[
        pltpu.VMEM((CPP, CHUNK), jnp.int32),
        pltpu.VMEM((128,), jnp.float32),
        pltpu.VMEM_SHARED((NPC,), jnp.float32),
    ],
)
def _sc_count(col_hbm, cnt_hbm, idx_v, ones_v, cnt_sh):
    c = lax.axis_index("c")
    s = lax.axis_index("s")
    w = c * NS + s
    for k in range(8):
        ones_v[pl.ds(k * 16, 16)] = jnp.zeros((16,), jnp.float32)
    for t in range(RPSC // 128):
        pltpu.sync_copy(ones_v, cnt_sh.at[pl.ds(s * RPSC + t * 128, 128)])
    for k in range(8):
        ones_v[pl.ds(k * 16, 16)] = jnp.ones((16,), jnp.float32)
    plsc.subcore_barrier()

    def phase(h, carry):
        pltpu.sync_copy(col_hbm.at[w, h], idx_v)

        def body(j, carry2):
            pltpu.sync_copy(ones_v.at[pl.ds(0, CHUNK)],
                            cnt_sh.at[idx_v.at[j]], add=True)
            return carry2

        lax.fori_loop(0, CPP, body, 0)
        return carry

    lax.fori_loop(0, NPH, phase, 0)
    plsc.subcore_barrier()
    pltpu.sync_copy(cnt_sh.at[pl.ds(s * RPSC, RPSC)],
                    cnt_hbm.at[c, pl.ds(s * RPSC, RPSC)])


@functools.partial(
    pl.kernel,
    out_type=jax.ShapeDtypeStruct((NC, NPAD, D), jnp.float32),
    mesh=_mesh,
    scratch_types=[
        pltpu.VMEM((CPP, CHUNK), jnp.int32),
        pltpu.VMEM((CPP, CHUNK), jnp.int32),
        pltpu.VMEM((3, CHUNK, D), jnp.float32),
        pltpu.VMEM_SHARED((NPAD, D), jnp.float32),
        pltpu.SemaphoreType.DMA((3,)),
    ],
)
def _sc_aggregate(xs_hbm, row_hbm, col_hbm, agg_hbm,
                  row_v, col_v, buf2, agg_sh, sem2):
    c = lax.axis_index("c")
    s = lax.axis_index("s")
    w = c * NS + s

    def zbody(i, carry):
        for k in range(D // 16):
            buf2[0, i, pl.ds(k * 16, 16)] = jnp.zeros((16,), jnp.float32)
        return carry

    lax.fori_loop(0, CHUNK, zbody, 0)
    for t in range(6):
        pltpu.sync_copy(buf2.at[0, pl.ds(0, 96)],
                        agg_sh.at[pl.ds(s * RPS + t * 96, 96)])
    pltpu.sync_copy(buf2.at[0, pl.ds(0, 56)],
                    agg_sh.at[pl.ds(s * RPS + 576, 56)])
    plsc.subcore_barrier()

    def phase(h, carry):
        pltpu.sync_copy(row_hbm.at[w, h], row_v)
        pltpu.sync_copy(col_hbm.at[w, h], col_v)

        def body(j, carry2):
            @pl.when(j < CPP)
            def _():
                b = j % 3
                pltpu.async_copy(xs_hbm.at[row_v.at[j]], buf2.at[b],
                                 sem2.at[b])

            @pl.when(j > 1)
            def _():
                p = (j - 2) % 3
                pltpu.make_async_copy(xs_hbm.at[row_v.at[0]],
                                      buf2.at[p], sem2.at[p]).wait()
                pltpu.sync_copy(buf2.at[p], agg_sh.at[col_v.at[j - 2]],
                                add=True)

            return carry2

        lax.fori_loop(0, CPP + 2, body, 0)
        return carry

    lax.fori_loop(0, NPH, phase, 0)
    plsc.subcore_barrier()
    pltpu.sync_copy(agg_sh.at[pl.ds(s * RPS, RPS)],
                    agg_hbm.at[c, pl.ds(s * RPS, RPS)])


def _prep_body(dinv_ref, x_ref, xs_ref):
    xs_ref[...] = x_ref[...] * dinv_ref[...]


_prep = pl.pallas_call(
    _prep_body,
    grid=(10,),
    in_specs=[pl.BlockSpec((N // 10, 1), lambda i: (i, 0)),
              pl.BlockSpec((N // 10, D), lambda i: (i, 0))],
    out_specs=pl.BlockSpec((N // 10, D), lambda i: (i, 0)),
    out_shape=jax.ShapeDtypeStruct((N, D), jnp.float32),
)


def _combine_body(agg_ref, d1_ref, d2_ref, x_ref, x0_ref, wp_ref, out_ref):
    b = agg_ref[0] + agg_ref[1]
    support = d1_ref[...] * b + d2_ref[...] * x_ref[...] + ALPHA * x0_ref[...]
    out_ref[...] = jnp.dot(support, wp_ref[...],
                           preferred_element_type=jnp.float32)


_combine = pl.pallas_call(
    _combine_body,
    grid=(10,),
    in_specs=[pl.BlockSpec((NC, N // 10, D), lambda i: (0, i, 0)),
              pl.BlockSpec((N // 10, 1), lambda i: (i, 0)),
              pl.BlockSpec((N // 10, 1), lambda i: (i, 0)),
              pl.BlockSpec((N // 10, D), lambda i: (i, 0)),
              pl.BlockSpec((N // 10, D), lambda i: (i, 0)),
              pl.BlockSpec((D, D), lambda i: (0, 0))],
    out_specs=pl.BlockSpec((N // 10, D), lambda i: (i, 0)),
    out_shape=jax.ShapeDtypeStruct((N, D), jnp.float32),
)


def kernel(x, edge_index, x0, W):
    row_p = edge_index[0].reshape(NW, NPH, CPP, CHUNK)
    col_p = edge_index[1].reshape(NW, NPH, CPP, CHUNK)

    cnt2 = _sc_count(col_p)
    deg = cnt2[0, :N] + cnt2[1, :N] + 1.0
    d = lax.rsqrt(deg).reshape(N, 1)

    xs = _prep(d, x)
    agg2 = _sc_aggregate(xs, row_p, col_p)

    d1 = (1.0 - ALPHA) * d
    d2 = d1 * d
    Wp = (1.0 - BETA) * jnp.eye(D, dtype=jnp.float32) + BETA * W
    return _combine(agg2, d1, d2, x, x0, Wp)
